# baseline (speedup 1.0000x reference)
import functools

import jax
import jax.numpy as jnp
from jax import lax
from jax.experimental import pallas as pl
from jax.experimental.pallas import tpu as pltpu
from jax.experimental.pallas import tpu_sc as plsc

N_NODES = 10000
N_EDGES = 320000
D_FEAT = 128
HIDDEN = 64
BETA = 0.1

NC = 2
NS = 16
L = 16
NW = NC * NS
EDGES_PER_TILE = N_EDGES // NW
CHUNK = 400
N_CHUNKS = EDGES_PER_TILE // CHUNK
GATE_COLS = 128
GATE_ROWS = N_EDGES // GATE_COLS
PARAM_PAD = 72


def _tc_prep(x_ref, w1a_ref, w1b_ref, b1_ref, la_ref, u_ref,
             a_ref, b_ref, g_ref):
    xv = x_ref[...]
    a_ref[...] = jnp.dot(xv, w1a_ref[...],
                         preferred_element_type=jnp.float32) + b1_ref[...]
    b_ref[...] = jnp.dot(xv, w1b_ref[...],
                         preferred_element_type=jnp.float32)
    uv = u_ref[...]
    z = (la_ref[...] + jnp.log(uv) - jnp.log(1.0 - uv)) * (1.0 / BETA)
    g_ref[...] = jax.nn.sigmoid(z)


def _sc_edge(a_hbm, b_hbm, i_hbm, j_hbm, g_hbm, p_hbm, out_hbm,
             idx_i, idx_j, sa, sb, gv, ov, pv, sem_a, sem_b):
    wid = lax.axis_index("s") * NC + lax.axis_index("c")
    pltpu.sync_copy(p_hbm, pv)
    w2_vecs = [pv[pl.ds(k * L, L)] for k in range(HIDDEN // L)]
    b2_s = pv[pl.ds(HIDDEN - L + 8, L)][8]

    def chunk_body(c, carry):
        base = wid * EDGES_PER_TILE + c * CHUNK
        pltpu.sync_copy(i_hbm.at[pl.ds(base, CHUNK)], idx_i)
        pltpu.sync_copy(j_hbm.at[pl.ds(base, CHUNK)], idx_j)
        pltpu.sync_copy(g_hbm.at[pl.ds(base, CHUNK)], gv)
        ca = pltpu.async_copy(a_hbm.at[idx_i], sa, sem_a)
        cb = pltpu.async_copy(b_hbm.at[idx_j], sb, sem_b)
        ca.wait()
        cb.wait()

        lane = lax.iota(jnp.int32, L)

        def edge_body(t, carry2):
            e0 = t * L
            sims = jnp.zeros((L,), jnp.float32)
            for l in range(L):
                e = e0 + l
                acc = None
                for k in range(HIDDEN // L):
                    av = sa[e, pl.ds(k * L, L)]
                    bv = sb[e, pl.ds(k * L, L)]
                    term = jnp.maximum(av + bv, 0.0) * w2_vecs[k]
                    acc = term if acc is None else acc + term
                sims = jnp.where(lane == l, jnp.sum(acc), sims)
            ov[pl.ds(e0, L)] = (sims + b2_s) * gv[pl.ds(e0, L)]
            return carry2

        lax.fori_loop(0, CHUNK // L, edge_body, 0)
        pltpu.sync_copy(ov, out_hbm.at[pl.ds(base, CHUNK)])
        return carry

    lax.fori_loop(0, N_CHUNKS, chunk_body, 0)


_sc_edge_call = functools.partial(
    pl.kernel,
    out_type=jax.ShapeDtypeStruct((N_EDGES,), jnp.float32),
    mesh=plsc.VectorSubcoreMesh(core_axis_name="c", subcore_axis_name="s",
                                num_cores=NC, num_subcores=NS),
    scratch_types=[
        pltpu.VMEM((CHUNK,), jnp.int32),
        pltpu.VMEM((CHUNK,), jnp.int32),
        pltpu.VMEM((CHUNK, HIDDEN), jnp.float32),
        pltpu.VMEM((CHUNK, HIDDEN), jnp.float32),
        pltpu.VMEM((CHUNK,), jnp.float32),
        pltpu.VMEM((CHUNK,), jnp.float32),
        pltpu.VMEM((PARAM_PAD,), jnp.float32),
        pltpu.SemaphoreType.DMA,
        pltpu.SemaphoreType.DMA,
    ],
    compiler_params=pltpu.CompilerParams(use_tc_tiling_on_sc=False,
                                         needs_layout_passes=False),
)(_sc_edge)


def kernel(x, edge_index, edge_log_alpha, W1, b1, W2, b2, u):
    w1a_t = W1[:, :D_FEAT].T
    w1b_t = W1[:, D_FEAT:].T
    b1_2d = b1.reshape(1, HIDDEN)
    la_2d = edge_log_alpha.reshape(GATE_ROWS, GATE_COLS)
    u_2d = u.reshape(GATE_ROWS, GATE_COLS)

    tables_a, tables_b, gates_2d = pl.pallas_call(
        _tc_prep,
        out_shape=[
            jax.ShapeDtypeStruct((N_NODES, HIDDEN), jnp.float32),
            jax.ShapeDtypeStruct((N_NODES, HIDDEN), jnp.float32),
            jax.ShapeDtypeStruct((GATE_ROWS, GATE_COLS), jnp.float32),
        ],
    )(x, w1a_t, w1b_t, b1_2d, la_2d, u_2d)

    params = jnp.concatenate(
        [W2[0], b2, jnp.zeros((PARAM_PAD - HIDDEN - 1,), jnp.float32)])

    out = _sc_edge_call(tables_a, tables_b, edge_index[0], edge_index[1],
                        gates_2d.reshape(N_EDGES), params)
    return out


# speedup vs baseline: 1.4968x; 1.4968x over previous
import functools

import jax
import jax.numpy as jnp
from jax import lax
from jax.experimental import pallas as pl
from jax.experimental.pallas import tpu as pltpu
from jax.experimental.pallas import tpu_sc as plsc

N_NODES = 10000
N_EDGES = 320000
D_FEAT = 128
HIDDEN = 64
BETA = 0.1

NC = 2
NS = 16
L = 16
NW = NC * NS
EDGES_PER_TILE = N_EDGES // NW
CHUNK = 400
N_CHUNKS = EDGES_PER_TILE // CHUNK
GATE_COLS = 128
GATE_ROWS = N_EDGES // GATE_COLS
PARAM_PAD = 72


def _tc_prep(x_ref, w1a_ref, w1b_ref, b1_ref, la_ref, u_ref,
             a_ref, b_ref, g_ref):
    xv = x_ref[...]
    a_ref[...] = jnp.dot(xv, w1a_ref[...],
                         preferred_element_type=jnp.float32) + b1_ref[...]
    b_ref[...] = jnp.dot(xv, w1b_ref[...],
                         preferred_element_type=jnp.float32)
    uv = u_ref[...]
    z = (la_ref[...] + jnp.log(uv) - jnp.log(1.0 - uv)) * (1.0 / BETA)
    g_ref[...] = jax.nn.sigmoid(z)


def _sc_edge(a_hbm, b_hbm, i_hbm, j_hbm, g_hbm, p_hbm, out_hbm,
             idx_i0, idx_j0, sa0, sb0, gv0, ov0,
             idx_i1, idx_j1, sa1, sb1, gv1, ov1,
             pv, sem_a0, sem_b0, sem_a1, sem_b1, sem_s0, sem_s1):
    wid = lax.axis_index("s") * NC + lax.axis_index("c")
    tile_base = wid * EDGES_PER_TILE
    pltpu.sync_copy(p_hbm, pv)
    w2_vecs = [pv[pl.ds(k * L, L)] for k in range(HIDDEN // L)]
    b2_s = pv[pl.ds(HIDDEN - L + 8, L)][8]
    lane = lax.iota(jnp.int32, L)

    bufs = [
        (idx_i0, idx_j0, sa0, sb0, gv0, ov0, sem_a0, sem_b0, sem_s0),
        (idx_i1, idx_j1, sa1, sb1, gv1, ov1, sem_a1, sem_b1, sem_s1),
    ]

    def stage3(buf, c):
        idx_i, idx_j, sa, sb, gv, ov, sem_a, sem_b, sem_s = buf
        base = tile_base + c * CHUNK
        c1 = pltpu.async_copy(i_hbm.at[pl.ds(base, CHUNK)], idx_i, sem_s)
        c2 = pltpu.async_copy(j_hbm.at[pl.ds(base, CHUNK)], idx_j, sem_s)
        c3 = pltpu.async_copy(g_hbm.at[pl.ds(base, CHUNK)], gv, sem_s)
        c1.wait()
        c2.wait()
        c3.wait()

    def fire_rows(buf):
        idx_i, idx_j, sa, sb, gv, ov, sem_a, sem_b, sem_s = buf
        pltpu.async_copy(a_hbm.at[idx_i], sa, sem_a)
        pltpu.async_copy(b_hbm.at[idx_j], sb, sem_b)

    def wait_rows(buf):
        idx_i, idx_j, sa, sb, gv, ov, sem_a, sem_b, sem_s = buf
        pltpu.make_async_copy(a_hbm.at[idx_i], sa, sem_a).wait()
        pltpu.make_async_copy(b_hbm.at[idx_j], sb, sem_b).wait()

    def compute(buf, c):
        idx_i, idx_j, sa, sb, gv, ov, sem_a, sem_b, sem_s = buf
        base = tile_base + c * CHUNK

        def edge_body(t, carry2):
            e0 = t * L
            sims = jnp.zeros((L,), jnp.float32)
            for l in range(L):
                e = e0 + l
                acc = None
                for k in range(HIDDEN // L):
                    av = sa[e, pl.ds(k * L, L)]
                    bv = sb[e, pl.ds(k * L, L)]
                    term = jnp.maximum(av + bv, 0.0) * w2_vecs[k]
                    acc = term if acc is None else acc + term
                sims = jnp.where(lane == l, jnp.sum(acc), sims)
            ov[pl.ds(e0, L)] = (sims + b2_s) * gv[pl.ds(e0, L)]
            return carry2

        lax.fori_loop(0, CHUNK // L, edge_body, 0)
        pltpu.sync_copy(ov, out_hbm.at[pl.ds(base, CHUNK)])

    stage3(bufs[0], 0)
    fire_rows(bufs[0])

    def pair_body(p, carry):
        c0 = 2 * p
        wait_rows(bufs[0])
        stage3(bufs[1], c0 + 1)
        fire_rows(bufs[1])
        compute(bufs[0], c0)
        wait_rows(bufs[1])
        stage3(bufs[0], c0 + 2)
        fire_rows(bufs[0])
        compute(bufs[1], c0 + 1)
        return carry

    lax.fori_loop(0, (N_CHUNKS - 1) // 2, pair_body, 0)
    wait_rows(bufs[0])
    compute(bufs[0], N_CHUNKS - 1)


_sc_edge_call = functools.partial(
    pl.kernel,
    out_type=jax.ShapeDtypeStruct((N_EDGES,), jnp.float32),
    mesh=plsc.VectorSubcoreMesh(core_axis_name="c", subcore_axis_name="s",
                                num_cores=NC, num_subcores=NS),
    scratch_types=(
        [
            pltpu.VMEM((CHUNK,), jnp.int32),
            pltpu.VMEM((CHUNK,), jnp.int32),
            pltpu.VMEM((CHUNK, HIDDEN), jnp.float32),
            pltpu.VMEM((CHUNK, HIDDEN), jnp.float32),
            pltpu.VMEM((CHUNK,), jnp.float32),
            pltpu.VMEM((CHUNK,), jnp.float32),
        ] * 2
        + [pltpu.VMEM((PARAM_PAD,), jnp.float32)]
        + [pltpu.SemaphoreType.DMA] * 6
    ),
    compiler_params=pltpu.CompilerParams(use_tc_tiling_on_sc=False,
                                         needs_layout_passes=False),
)(_sc_edge)


def kernel(x, edge_index, edge_log_alpha, W1, b1, W2, b2, u):
    w1a_t = W1[:, :D_FEAT].T
    w1b_t = W1[:, D_FEAT:].T
    b1_2d = b1.reshape(1, HIDDEN)
    la_2d = edge_log_alpha.reshape(GATE_ROWS, GATE_COLS)
    u_2d = u.reshape(GATE_ROWS, GATE_COLS)

    tables_a, tables_b, gates_2d = pl.pallas_call(
        _tc_prep,
        out_shape=[
            jax.ShapeDtypeStruct((N_NODES, HIDDEN), jnp.float32),
            jax.ShapeDtypeStruct((N_NODES, HIDDEN), jnp.float32),
            jax.ShapeDtypeStruct((GATE_ROWS, GATE_COLS), jnp.float32),
        ],
    )(x, w1a_t, w1b_t, b1_2d, la_2d, u_2d)

    params = jnp.concatenate(
        [W2[0], b2, jnp.zeros((PARAM_PAD - HIDDEN - 1,), jnp.float32)])

    out = _sc_edge_call(tables_a, tables_b, edge_index[0], edge_index[1],
                        gates_2d.reshape(N_EDGES), params)
    return out


# speedup vs baseline: 1.7492x; 1.1687x over previous
import functools

import jax
import jax.numpy as jnp
from jax import lax
from jax.experimental import pallas as pl
from jax.experimental.pallas import tpu as pltpu
from jax.experimental.pallas import tpu_sc as plsc

N_NODES = 10000
N_EDGES = 320000
D_FEAT = 128
HIDDEN = 64
BETA = 0.1

NC = 2
NS = 16
L = 16
NW = NC * NS
EDGES_PER_TILE = N_EDGES // NW
CHUNK = 400
N_CHUNKS = EDGES_PER_TILE // CHUNK
GATE_COLS = 128
GATE_ROWS = N_EDGES // GATE_COLS
PARAM_PAD = 72


def _tc_prep(x_ref, w1a_ref, w1b_ref, b1_ref, la_ref, u_ref,
             a_ref, b_ref, g_ref):
    xv = x_ref[...]
    a_ref[...] = (jnp.dot(xv, w1a_ref[...],
                          preferred_element_type=jnp.float32)
                  + b1_ref[...]).astype(jnp.bfloat16)
    b_ref[...] = jnp.dot(xv, w1b_ref[...],
                         preferred_element_type=jnp.float32).astype(jnp.bfloat16)
    uv = u_ref[...]
    z = (la_ref[...] + jnp.log(uv) - jnp.log(1.0 - uv)) * (1.0 / BETA)
    g_ref[...] = jax.nn.sigmoid(z)


def _sc_edge(a_hbm, b_hbm, i_hbm, j_hbm, g_hbm, p_hbm, out_hbm,
             idx_i0, idx_j0, sa0, sb0, gv0, ov0,
             idx_i1, idx_j1, sa1, sb1, gv1, ov1,
             pv, sem_a0, sem_b0, sem_a1, sem_b1, sem_s0, sem_s1):
    wid = lax.axis_index("s") * NC + lax.axis_index("c")
    tile_base = wid * EDGES_PER_TILE
    pltpu.sync_copy(p_hbm, pv)
    w2_vecs = [pv[pl.ds(k * L, L)] for k in range(HIDDEN // L)]
    b2_s = pv[pl.ds(HIDDEN - L + 8, L)][8]
    lane = lax.iota(jnp.int32, L)

    bufs = [
        (idx_i0, idx_j0, sa0, sb0, gv0, ov0, sem_a0, sem_b0, sem_s0),
        (idx_i1, idx_j1, sa1, sb1, gv1, ov1, sem_a1, sem_b1, sem_s1),
    ]

    def stage3(buf, c):
        idx_i, idx_j, sa, sb, gv, ov, sem_a, sem_b, sem_s = buf
        base = tile_base + c * CHUNK
        c1 = pltpu.async_copy(i_hbm.at[pl.ds(base, CHUNK)], idx_i, sem_s)
        c2 = pltpu.async_copy(j_hbm.at[pl.ds(base, CHUNK)], idx_j, sem_s)
        c3 = pltpu.async_copy(g_hbm.at[pl.ds(base, CHUNK)], gv, sem_s)
        c1.wait()
        c2.wait()
        c3.wait()

    def fire_rows(buf):
        idx_i, idx_j, sa, sb, gv, ov, sem_a, sem_b, sem_s = buf
        pltpu.async_copy(a_hbm.at[idx_i], sa, sem_a)
        pltpu.async_copy(b_hbm.at[idx_j], sb, sem_b)

    def wait_rows(buf):
        idx_i, idx_j, sa, sb, gv, ov, sem_a, sem_b, sem_s = buf
        pltpu.make_async_copy(a_hbm.at[idx_i], sa, sem_a).wait()
        pltpu.make_async_copy(b_hbm.at[idx_j], sb, sem_b).wait()

    def compute(buf, c):
        idx_i, idx_j, sa, sb, gv, ov, sem_a, sem_b, sem_s = buf
        base = tile_base + c * CHUNK

        def edge_body(t, carry2):
            e0 = t * L
            sims = jnp.zeros((L,), jnp.float32)
            for l in range(L):
                e = e0 + l
                acc = None
                for k in range(HIDDEN // (2 * L)):
                    ae, ao = plsc.unpack(
                        sa[e, pl.ds(2 * L * k, 2 * L)],
                        format=plsc.PackFormat.INTERLEAVED)
                    be, bo = plsc.unpack(
                        sb[e, pl.ds(2 * L * k, 2 * L)],
                        format=plsc.PackFormat.INTERLEAVED)
                    term = (jnp.maximum(ae + be, 0.0) * w2_vecs[2 * k]
                            + jnp.maximum(ao + bo, 0.0) * w2_vecs[2 * k + 1])
                    acc = term if acc is None else acc + term
                sims = jnp.where(lane == l, jnp.sum(acc), sims)
            ov[pl.ds(e0, L)] = (sims + b2_s) * gv[pl.ds(e0, L)]
            return carry2

        lax.fori_loop(0, CHUNK // L, edge_body, 0)
        pltpu.sync_copy(ov, out_hbm.at[pl.ds(base, CHUNK)])

    stage3(bufs[0], 0)
    fire_rows(bufs[0])

    def pair_body(p, carry):
        c0 = 2 * p
        wait_rows(bufs[0])
        stage3(bufs[1], c0 + 1)
        fire_rows(bufs[1])
        compute(bufs[0], c0)
        wait_rows(bufs[1])
        stage3(bufs[0], c0 + 2)
        fire_rows(bufs[0])
        compute(bufs[1], c0 + 1)
        return carry

    lax.fori_loop(0, (N_CHUNKS - 1) // 2, pair_body, 0)
    wait_rows(bufs[0])
    compute(bufs[0], N_CHUNKS - 1)


_sc_edge_call = functools.partial(
    pl.kernel,
    out_type=jax.ShapeDtypeStruct((N_EDGES,), jnp.float32),
    mesh=plsc.VectorSubcoreMesh(core_axis_name="c", subcore_axis_name="s",
                                num_cores=NC, num_subcores=NS),
    scratch_types=(
        [
            pltpu.VMEM((CHUNK,), jnp.int32),
            pltpu.VMEM((CHUNK,), jnp.int32),
            pltpu.VMEM((CHUNK, HIDDEN), jnp.bfloat16),
            pltpu.VMEM((CHUNK, HIDDEN), jnp.bfloat16),
            pltpu.VMEM((CHUNK,), jnp.float32),
            pltpu.VMEM((CHUNK,), jnp.float32),
        ] * 2
        + [pltpu.VMEM((PARAM_PAD,), jnp.float32)]
        + [pltpu.SemaphoreType.DMA] * 6
    ),
    compiler_params=pltpu.CompilerParams(use_tc_tiling_on_sc=False,
                                         needs_layout_passes=False),
)(_sc_edge)


def kernel(x, edge_index, edge_log_alpha, W1, b1, W2, b2, u):
    w1a_t = W1[:, :D_FEAT].T
    w1b_t = W1[:, D_FEAT:].T
    b1_2d = b1.reshape(1, HIDDEN)
    la_2d = edge_log_alpha.reshape(GATE_ROWS, GATE_COLS)
    u_2d = u.reshape(GATE_ROWS, GATE_COLS)

    tables_a, tables_b, gates_2d = pl.pallas_call(
        _tc_prep,
        out_shape=[
            jax.ShapeDtypeStruct((N_NODES, HIDDEN), jnp.bfloat16),
            jax.ShapeDtypeStruct((N_NODES, HIDDEN), jnp.bfloat16),
            jax.ShapeDtypeStruct((GATE_ROWS, GATE_COLS), jnp.float32),
        ],
    )(x, w1a_t, w1b_t, b1_2d, la_2d, u_2d)

    w2 = W2[0]
    w2_perm = jnp.concatenate(
        [w2[0:32:2], w2[1:32:2], w2[32:64:2], w2[33:64:2]])
    params = jnp.concatenate(
        [w2_perm, b2, jnp.zeros((PARAM_PAD - HIDDEN - 1,), jnp.float32)])

    out = _sc_edge_call(tables_a, tables_b, edge_index[0], edge_index[1],
                        gates_2d.reshape(N_EDGES), params)
    return out


# speedup vs baseline: 1.7590x; 1.0056x over previous
import functools

import jax
import jax.numpy as jnp
from jax import lax
from jax.experimental import pallas as pl
from jax.experimental.pallas import tpu as pltpu
from jax.experimental.pallas import tpu_sc as plsc

N_NODES = 10000
N_EDGES = 320000
D_FEAT = 128
HIDDEN = 64
BETA = 0.1

NC = 2
NS = 16
L = 16
NW = NC * NS
EDGES_PER_TILE = N_EDGES // NW
CHUNK = 400
N_CHUNKS = EDGES_PER_TILE // CHUNK
GATE_COLS = 128
GATE_ROWS = N_EDGES // GATE_COLS
PARAM_PAD = 72


def _tc_prep(x_ref, w1a_ref, w1b_ref, b1_ref, la_ref, u_ref,
             a_ref, b_ref, g_ref):
    xv = x_ref[...]
    a_ref[...] = (jnp.dot(xv, w1a_ref[...],
                          preferred_element_type=jnp.float32)
                  + b1_ref[...]).astype(jnp.bfloat16)
    b_ref[...] = jnp.dot(xv, w1b_ref[...],
                         preferred_element_type=jnp.float32).astype(jnp.bfloat16)
    uv = u_ref[...]
    z = (la_ref[...] + jnp.log(uv) - jnp.log(1.0 - uv)) * (1.0 / BETA)
    g_ref[...] = jax.nn.sigmoid(z)


def _sc_edge(a_hbm, b_hbm, i_hbm, j_hbm, g_hbm, p_hbm, out_hbm,
             idx_i0, idx_j0, sa0, sb0, gv0, ov0,
             idx_i1, idx_j1, sa1, sb1, gv1, ov1,
             pv, sem_a0, sem_b0, sem_a1, sem_b1, sem_s0, sem_s1):
    wid = lax.axis_index("s") * NC + lax.axis_index("c")
    tile_base = wid * EDGES_PER_TILE
    pltpu.sync_copy(p_hbm, pv)
    w2_vecs = [pv[pl.ds(k * L, L)] for k in range(HIDDEN // L)]
    b2_s = pv[pl.ds(HIDDEN - L + 8, L)][8]
    lane = lax.iota(jnp.int32, L)

    bufs = [
        (idx_i0, idx_j0, sa0, sb0, gv0, ov0, sem_a0, sem_b0, sem_s0),
        (idx_i1, idx_j1, sa1, sb1, gv1, ov1, sem_a1, sem_b1, sem_s1),
    ]

    def stage3_issue(buf, c):
        idx_i, idx_j, sa, sb, gv, ov, sem_a, sem_b, sem_s = buf
        base = tile_base + c * CHUNK
        pltpu.async_copy(i_hbm.at[pl.ds(base, CHUNK)], idx_i, sem_s)
        pltpu.async_copy(j_hbm.at[pl.ds(base, CHUNK)], idx_j, sem_s)
        pltpu.async_copy(g_hbm.at[pl.ds(base, CHUNK)], gv, sem_s)

    def stage3_wait(buf, c):
        idx_i, idx_j, sa, sb, gv, ov, sem_a, sem_b, sem_s = buf
        base = tile_base + c * CHUNK
        pltpu.make_async_copy(i_hbm.at[pl.ds(base, CHUNK)], idx_i,
                              sem_s).wait()
        pltpu.make_async_copy(j_hbm.at[pl.ds(base, CHUNK)], idx_j,
                              sem_s).wait()
        pltpu.make_async_copy(g_hbm.at[pl.ds(base, CHUNK)], gv,
                              sem_s).wait()

    def fire_rows(buf):
        idx_i, idx_j, sa, sb, gv, ov, sem_a, sem_b, sem_s = buf
        pltpu.async_copy(a_hbm.at[idx_i], sa, sem_a)
        pltpu.async_copy(b_hbm.at[idx_j], sb, sem_b)

    def wait_rows(buf):
        idx_i, idx_j, sa, sb, gv, ov, sem_a, sem_b, sem_s = buf
        pltpu.make_async_copy(a_hbm.at[idx_i], sa, sem_a).wait()
        pltpu.make_async_copy(b_hbm.at[idx_j], sb, sem_b).wait()

    def compute(buf, c):
        idx_i, idx_j, sa, sb, gv, ov, sem_a, sem_b, sem_s = buf
        base = tile_base + c * CHUNK

        def edge_body(t, carry2):
            e0 = t * L
            sims = jnp.zeros((L,), jnp.float32)
            for l in range(L):
                e = e0 + l
                acc = None
                for k in range(HIDDEN // (2 * L)):
                    ae, ao = plsc.unpack(
                        sa[e, pl.ds(2 * L * k, 2 * L)],
                        format=plsc.PackFormat.INTERLEAVED)
                    be, bo = plsc.unpack(
                        sb[e, pl.ds(2 * L * k, 2 * L)],
                        format=plsc.PackFormat.INTERLEAVED)
                    term = (jnp.maximum(ae + be, 0.0) * w2_vecs[2 * k]
                            + jnp.maximum(ao + bo, 0.0) * w2_vecs[2 * k + 1])
                    acc = term if acc is None else acc + term
                sims = jnp.where(lane == l, jnp.sum(acc), sims)
            ov[pl.ds(e0, L)] = (sims + b2_s) * gv[pl.ds(e0, L)]
            return carry2

        lax.fori_loop(0, CHUNK // L, edge_body, 0)
        pltpu.sync_copy(ov, out_hbm.at[pl.ds(base, CHUNK)])

    stage3_issue(bufs[0], 0)
    stage3_wait(bufs[0], 0)
    fire_rows(bufs[0])

    def pair_body(p, carry):
        c0 = 2 * p
        stage3_issue(bufs[1], c0 + 1)
        wait_rows(bufs[0])
        stage3_wait(bufs[1], c0 + 1)
        fire_rows(bufs[1])
        compute(bufs[0], c0)
        stage3_issue(bufs[0], c0 + 2)
        wait_rows(bufs[1])
        stage3_wait(bufs[0], c0 + 2)
        fire_rows(bufs[0])
        compute(bufs[1], c0 + 1)
        return carry

    lax.fori_loop(0, (N_CHUNKS - 1) // 2, pair_body, 0)
    wait_rows(bufs[0])
    compute(bufs[0], N_CHUNKS - 1)


_sc_edge_call = functools.partial(
    pl.kernel,
    out_type=jax.ShapeDtypeStruct((N_EDGES,), jnp.float32),
    mesh=plsc.VectorSubcoreMesh(core_axis_name="c", subcore_axis_name="s",
                                num_cores=NC, num_subcores=NS),
    scratch_types=(
        [
            pltpu.VMEM((CHUNK,), jnp.int32),
            pltpu.VMEM((CHUNK,), jnp.int32),
            pltpu.VMEM((CHUNK, HIDDEN), jnp.bfloat16),
            pltpu.VMEM((CHUNK, HIDDEN), jnp.bfloat16),
            pltpu.VMEM((CHUNK,), jnp.float32),
            pltpu.VMEM((CHUNK,), jnp.float32),
        ] * 2
        + [pltpu.VMEM((PARAM_PAD,), jnp.float32)]
        + [pltpu.SemaphoreType.DMA] * 6
    ),
    compiler_params=pltpu.CompilerParams(use_tc_tiling_on_sc=False,
                                         needs_layout_passes=False),
)(_sc_edge)


def kernel(x, edge_index, edge_log_alpha, W1, b1, W2, b2, u):
    w1a_t = W1[:, :D_FEAT].T
    w1b_t = W1[:, D_FEAT:].T
    b1_2d = b1.reshape(1, HIDDEN)
    la_2d = edge_log_alpha.reshape(GATE_ROWS, GATE_COLS)
    u_2d = u.reshape(GATE_ROWS, GATE_COLS)

    tables_a, tables_b, gates_2d = pl.pallas_call(
        _tc_prep,
        out_shape=[
            jax.ShapeDtypeStruct((N_NODES, HIDDEN), jnp.bfloat16),
            jax.ShapeDtypeStruct((N_NODES, HIDDEN), jnp.bfloat16),
            jax.ShapeDtypeStruct((GATE_ROWS, GATE_COLS), jnp.float32),
        ],
    )(x, w1a_t, w1b_t, b1_2d, la_2d, u_2d)

    w2 = W2[0]
    w2_perm = jnp.concatenate(
        [w2[0:32:2], w2[1:32:2], w2[32:64:2], w2[33:64:2]])
    params = jnp.concatenate(
        [w2_perm, b2, jnp.zeros((PARAM_PAD - HIDDEN - 1,), jnp.float32)])

    out = _sc_edge_call(tables_a, tables_b, edge_index[0], edge_index[1],
                        gates_2d.reshape(N_EDGES), params)
    return out
